# Initial kernel scaffold; baseline (speedup 1.0000x reference)
import jax
import jax.numpy as jnp
from jax.experimental import pallas as pl


def kernel(node_states, segment_ids):
    raise NotImplementedError("write your pallas kernel here")


# speedup vs baseline: 5.3321x; 5.3321x over previous
import functools

import jax
import jax.numpy as jnp
from jax.experimental import pallas as pl
from jax.experimental.pallas import tpu as pltpu

NUM_SEGMENTS = 1024
BLOCK_R = 2048


def _pool_block(ids_ref, x_ref, out_ref, *, n_valid, block_r):
    i = pl.program_id(0)

    @pl.when(i == 0)
    def _():
        out_ref[...] = jnp.zeros_like(out_ref)

    ids = ids_ref[0, 0, :]
    seg_iota = jax.lax.broadcasted_iota(jnp.int32, (NUM_SEGMENTS, block_r), 0)
    one_hot = (seg_iota == ids[None, :]).astype(jnp.bfloat16)
    row = i * block_r + jax.lax.broadcasted_iota(jnp.int32, (block_r, 128), 0)
    x = jnp.where(row < n_valid, x_ref[...], 0.0).astype(jnp.bfloat16)
    out_ref[...] += jnp.dot(one_hot, x, preferred_element_type=jnp.float32)


def kernel(node_states, segment_ids):
    n, h = node_states.shape
    num_blocks = pl.cdiv(n, BLOCK_R)
    n_pad = num_blocks * BLOCK_R
    ids = jnp.full((n_pad,), NUM_SEGMENTS, jnp.int32)
    ids = ids.at[:n].set(segment_ids.astype(jnp.int32))
    ids = ids.reshape(num_blocks, 1, BLOCK_R)

    return pl.pallas_call(
        functools.partial(_pool_block, n_valid=n, block_r=BLOCK_R),
        grid=(num_blocks,),
        in_specs=[
            pl.BlockSpec((1, 1, BLOCK_R), lambda i: (i, 0, 0)),
            pl.BlockSpec((BLOCK_R, h), lambda i: (i, 0)),
        ],
        out_specs=pl.BlockSpec((NUM_SEGMENTS, h), lambda i: (0, 0)),
        out_shape=jax.ShapeDtypeStruct((NUM_SEGMENTS, h), jnp.float32),
        compiler_params=pltpu.CompilerParams(
            dimension_semantics=("arbitrary",),
        ),
    )(ids, node_states)
